# baseline (speedup 1.0000x reference)
import functools

import jax
import jax.numpy as jnp
from jax import lax
from jax.experimental import pallas as pl
from jax.experimental.pallas import tpu as pltpu
from jax.experimental.pallas import tpu_sc as plsc

CHUNK = 128
NTILES = 32
IBLK = 28
ZROWS = 112


def _npad(n):
    rpt = -(-(-(-(n + 1) // 16)) // ZROWS) * ZROWS
    return 16 * rpt


def _sc_mesh():
    return plsc.VectorSubcoreMesh(core_axis_name="c", subcore_axis_name="s")


_SC_PARAMS = pltpu.CompilerParams(use_tc_tiling_on_sc=False)


def _matvec_sc(n_pad, cpad, cpt):
    rpt = n_pad // 16
    nblk = cpt // IBLK
    S = 7 if cpad <= 16 else 4
    L = S // 2
    assert IBLK % S == 0

    @functools.partial(
        pl.kernel,
        out_type=jax.ShapeDtypeStruct((2, n_pad, cpad), jnp.float32),
        mesh=_sc_mesh(),
        compiler_params=_SC_PARAMS,
        scratch_types=[
            pltpu.VMEM((2, IBLK, CHUNK), jnp.int32),
            pltpu.VMEM((S, CHUNK, cpad), jnp.float32),
            pltpu.VMEM_SHARED((n_pad, cpad), jnp.float32),
            [pltpu.SemaphoreType.DMA] * S,
            [pltpu.SemaphoreType.DMA] * S,
        ],
    )
    def mv(u_hbm, gs_hbm, out_hbm, gs_v, rows_v, acc, sg, ss):
        cid = lax.axis_index("c")
        sid = lax.axis_index("s")
        wid = cid * 16 + sid

        def _wait_g(j, b):
            pltpu.make_async_copy(u_hbm.at[gs_v.at[0, j]],
                                  rows_v.at[b], sg[b]).wait()

        def _wait_s(b):
            pltpu.make_async_copy(rows_v.at[b],
                                  acc.at[pl.ds(0, CHUNK)], ss[b]).wait()

        def _zfill(r, c):
            for j in range(cpad // 16):
                rows_v[0, r, pl.ds(16 * j, 16)] = jnp.zeros((16,), jnp.float32)
            return c

        lax.fori_loop(0, ZROWS, _zfill, 0)
        base = sid * rpt
        for j in range(rpt // ZROWS):
            pltpu.sync_copy(rows_v.at[0, pl.ds(0, ZROWS)],
                            acc.at[pl.ds(base + j * ZROWS, ZROWS)])
        plsc.subcore_barrier()

        def _block(ob, c):
            pltpu.sync_copy(gs_hbm.at[wid, ob], gs_v)
            for s in range(L):
                @pl.when(ob > 0)
                def _():
                    _wait_s(s)
                pltpu.async_copy(u_hbm.at[gs_v.at[0, s]], rows_v.at[s], sg[s])

            def _grp(p, c2):
                for b in range(S):
                    j = p * S + b
                    _wait_g(j, b)
                    pltpu.async_copy(rows_v.at[b], acc.at[gs_v.at[1, j]],
                                     ss[b], add=True)
                    b2 = (b + L) % S

                    @pl.when(j + L < IBLK)
                    def _():
                        if b < S - L:
                            @pl.when(ob + p > 0)
                            def _w():
                                _wait_s(b2)
                        else:
                            _wait_s(b2)
                        pltpu.async_copy(u_hbm.at[gs_v.at[0, j + L]],
                                         rows_v.at[b2], sg[b2])
                return c2

            lax.fori_loop(0, IBLK // S, _grp, 0)
            return c

        lax.fori_loop(0, nblk, _block, 0)
        for t in range(S):
            _wait_s(t)
        plsc.subcore_barrier()
        pltpu.sync_copy(acc.at[pl.ds(base, rpt)],
                        out_hbm.at[cid, pl.ds(base, rpt)])

    return mv


def _degree_sc(n_pad, cpt):
    rpt = n_pad // 16

    @functools.partial(
        pl.kernel,
        out_type=jax.ShapeDtypeStruct((2, n_pad, 16), jnp.float32),
        mesh=_sc_mesh(),
        compiler_params=_SC_PARAMS,
        scratch_types=[
            pltpu.VMEM((IBLK, CHUNK), jnp.int32),
            pltpu.VMEM((CHUNK, 16), jnp.float32),
            pltpu.VMEM((ZROWS, 16), jnp.float32),
            pltpu.VMEM_SHARED((n_pad, 16), jnp.float32),
            pltpu.SemaphoreType.DMA,
        ],
    )
    def dg(s_hbm, out_hbm, s_v, ones_v, zb_v, acc, sem):
        cid = lax.axis_index("c")
        sid = lax.axis_index("s")
        wid = cid * 16 + sid

        def _ofill(r, c):
            ones_v[r, pl.ds(0, 16)] = jnp.ones((16,), jnp.float32)
            return c

        lax.fori_loop(0, CHUNK, _ofill, 0)

        def _zfill(r, c):
            zb_v[r, pl.ds(0, 16)] = jnp.zeros((16,), jnp.float32)
            return c

        lax.fori_loop(0, ZROWS, _zfill, 0)
        base = sid * rpt
        for j in range(rpt // ZROWS):
            pltpu.sync_copy(zb_v, acc.at[pl.ds(base + j * ZROWS, ZROWS)])
        plsc.subcore_barrier()

        def _outer(ob, c):
            pltpu.sync_copy(s_hbm.at[wid, pl.ds(ob * IBLK, IBLK)], s_v)
            for half in range(2):
                descs = [
                    pltpu.async_copy(ones_v, acc.at[s_v.at[half * (IBLK // 2) + j]],
                                     sem, add=True)
                    for j in range(IBLK // 2)
                ]
                for d in descs:
                    d.wait()
            return c

        lax.fori_loop(0, cpt // IBLK, _outer, 0)
        plsc.subcore_barrier()
        pltpu.sync_copy(acc.at[pl.ds(base, rpt)],
                        out_hbm.at[cid, pl.ds(base, rpt)])

    return dg


def _row_spec(rb, w):
    return pl.BlockSpec((rb, w), lambda i: (i, 0))


def _full_spec(shape):
    return pl.BlockSpec(shape, lambda i: tuple(0 for _ in shape))


def _prep_tc(n_pad, xp, da, db):
    rb = n_pad // 16

    def body(xp_ref, da_ref, db_ref, dis_ref, u0_ref):
        deg = da_ref[:, 0:1] + db_ref[:, 0:1]
        dis = jnp.where(deg > 0.0, lax.rsqrt(jnp.maximum(deg, 1e-30)), 0.0)
        dis_ref[...] = dis
        u0_ref[...] = dis * xp_ref[...]

    return pl.pallas_call(
        body,
        grid=(16,),
        in_specs=[_row_spec(rb, 16)] * 3,
        out_specs=[_row_spec(rb, 1), _row_spec(rb, 16)],
        out_shape=[jax.ShapeDtypeStruct((n_pad, 1), jnp.float32),
                   jax.ShapeDtypeStruct((n_pad, 16), jnp.float32)],
    )(xp, da, db)


def _cheb_step_tc(n_pad, cpad, first, sa, sb, dis, txm2):
    rb = n_pad // 16

    if first:
        def body(sa_ref, sb_ref, d_ref, tx_ref, u_ref):
            m = -(d_ref[...] * (sa_ref[...] + sb_ref[...]))
            tx_ref[...] = m
            u_ref[...] = d_ref[...] * m
        ins = [sa, sb, dis]
        in_specs = [_row_spec(rb, cpad)] * 2 + [_row_spec(rb, 1)]
    else:
        def body(sa_ref, sb_ref, d_ref, t2_ref, tx_ref, u_ref):
            m = -(d_ref[...] * (sa_ref[...] + sb_ref[...]))
            tx = 2.0 * m - t2_ref[...]
            tx_ref[...] = tx
            u_ref[...] = d_ref[...] * tx
        ins = [sa, sb, dis, txm2]
        in_specs = [_row_spec(rb, cpad)] * 2 + [_row_spec(rb, 1), _row_spec(rb, cpad)]

    return pl.pallas_call(
        body,
        grid=(16,),
        in_specs=in_specs,
        out_specs=[_row_spec(rb, cpad)] * 2,
        out_shape=[jax.ShapeDtypeStruct((n_pad, cpad), jnp.float32)] * 2,
    )(*ins)


def _cheb_step4_tc(n_pad, sa0, sb0, sa1, sb1, dis):
    rb = n_pad // 16

    def body(sa0_ref, sb0_ref, sa1_ref, sb1_ref, d_ref, tx_ref):
        s = jnp.concatenate([sa0_ref[...] + sb0_ref[...],
                             sa1_ref[...] + sb1_ref[...]], axis=1)
        tx_ref[...] = -(d_ref[...] * s)

    return pl.pallas_call(
        body,
        grid=(16,),
        in_specs=[_row_spec(rb, 32)] * 4 + [_row_spec(rb, 1)],
        out_specs=_row_spec(rb, 64),
        out_shape=jax.ShapeDtypeStruct((n_pad, 64), jnp.float32),
    )(sa0, sb0, sa1, sb1, dis)


def _layer_end_tc(n_pad, cpad_in, cout, cpad_out, split_u, txs, ws, bias, dis):
    rb = n_pad // 16
    K = len(txs)

    def body(*refs):
        tx_refs = refs[:K]
        w_refs = refs[K:2 * K]
        b_ref = refs[2 * K]
        d_ref = refs[2 * K + 1]
        outs = refs[2 * K + 2:]
        acc = b_ref[...]
        for k in range(K):
            acc = acc + jnp.dot(tx_refs[k][...], w_refs[k][...],
                                preferred_element_type=jnp.float32)
        z = 1.0 / (1.0 + jnp.exp(-acc[:, :cout]))
        ht = jnp.tanh(acc[:, cout:])
        h = jnp.maximum((1.0 - z) * ht, 0.0)
        if cpad_out > cout:
            h = jnp.concatenate(
                [h, jnp.zeros((rb, cpad_out - cout), jnp.float32)], axis=1)
        u = d_ref[...] * h
        outs[0][...] = h
        if split_u:
            outs[1][...] = u[:, :32]
            outs[2][...] = u[:, 32:]
        else:
            outs[1][...] = u

    in_specs = ([_row_spec(rb, cpad_in)] * K
                + [_full_spec((cpad_in, 2 * cout))] * K
                + [_full_spec((1, 2 * cout)), _row_spec(rb, 1)])
    if split_u:
        out_specs = [_row_spec(rb, cpad_out), _row_spec(rb, 32), _row_spec(rb, 32)]
        out_shape = [jax.ShapeDtypeStruct((n_pad, cpad_out), jnp.float32),
                     jax.ShapeDtypeStruct((n_pad, 32), jnp.float32),
                     jax.ShapeDtypeStruct((n_pad, 32), jnp.float32)]
    else:
        out_specs = [_row_spec(rb, cpad_out)] * 2
        out_shape = [jax.ShapeDtypeStruct((n_pad, cpad_out), jnp.float32)] * 2

    return pl.pallas_call(
        body,
        grid=(16,),
        in_specs=in_specs,
        out_specs=out_specs,
        out_shape=out_shape,
    )(*txs, *ws, bias, dis)


def _final_tc(n_pad, cout, tx0, tx1, w0, w1, bias, wl, bl):
    rb = n_pad // 16

    def body(t0, t1, w0_ref, w1_ref, b_ref, wl_ref, bl_ref, out_ref):
        acc = (b_ref[...]
               + jnp.dot(t0[...], w0_ref[...], preferred_element_type=jnp.float32)
               + jnp.dot(t1[...], w1_ref[...], preferred_element_type=jnp.float32))
        z = 1.0 / (1.0 + jnp.exp(-acc[:, :cout]))
        ht = jnp.tanh(acc[:, cout:])
        h = jnp.maximum((1.0 - z) * ht, 0.0)
        logits = jnp.dot(h, wl_ref[...], preferred_element_type=jnp.float32) + bl_ref[...]
        m = jnp.max(logits, axis=1, keepdims=True)
        e = jnp.exp(logits - m)
        out_ref[...] = e / jnp.sum(e, axis=1, keepdims=True)

    nc = wl.shape[1]
    return pl.pallas_call(
        body,
        grid=(16,),
        in_specs=[_row_spec(rb, 64)] * 2
        + [_full_spec((64, 2 * cout))] * 2
        + [_full_spec((1, 2 * cout)), _full_spec((cout, nc)), _full_spec((1, nc))],
        out_specs=_row_spec(rb, nc),
        out_shape=jax.ShapeDtypeStruct((n_pad, nc), jnp.float32),
    )(tx0, tx1, w0, w1, bias, wl, bl)


def _pad16(c):
    return -(-c // 16) * 16


def kernel(x, edge_index, params):
    n = x.shape[0]
    e = edge_index.shape[1]
    n_pad = _npad(n)
    cpt = IBLK * -(-e // (NTILES * CHUNK * IBLK))
    ep = NTILES * cpt * CHUNK

    nblk = cpt // IBLK
    row = edge_index[0]
    col = edge_index[1]
    pad_e = ep - e
    gather_pad = jnp.zeros((pad_e,), jnp.int32)
    scatter_pad = jnp.full((pad_e,), n, jnp.int32)
    g_row = jnp.concatenate([row, gather_pad]).reshape(NTILES, nblk, IBLK, CHUNK)
    s_col = jnp.concatenate([col, scatter_pad]).reshape(NTILES, nblk, IBLK, CHUNK)
    s_row = jnp.concatenate([row, scatter_pad]).reshape(NTILES, cpt, CHUNK)
    gs = jnp.stack([g_row, s_col], axis=2)

    degp = _degree_sc(n_pad, cpt)(s_row)
    xp = jnp.pad(x, ((0, n_pad - n), (0, _pad16(x.shape[1]) - x.shape[1])))
    dis, u = _prep_tc(n_pad, xp, degp[0], degp[1])

    h = xp
    n_layers = len(params["layers"])
    for li, lp in enumerate(params["layers"]):
        K, cin, cout = lp["xz"][0].shape
        cpad_in = _pad16(cin)
        ws = []
        for k in range(K):
            wk = jnp.concatenate([lp["xz"][0][k], lp["xh"][0][k]], axis=1)
            ws.append(jnp.pad(wk, ((0, cpad_in - cin), (0, 0))))
        bias = jnp.concatenate([lp["xz"][1] + lp["hz"][1],
                                lp["xh"][1] + lp["hh"][1]]).reshape(1, 2 * cout)

        if cpad_in <= 32:
            txs = [h]
            for k in range(1, K):
                sp = _matvec_sc(n_pad, cpad_in, cpt)(u, gs)
                tx, u = _cheb_step_tc(n_pad, cpad_in, k == 1, sp[0], sp[1],
                                      dis, None if k == 1 else txs[k - 2])
                txs.append(tx)
        else:
            ua, ub = u
            spa = _matvec_sc(n_pad, 32, cpt)(ua, gs)
            spb = _matvec_sc(n_pad, 32, cpt)(ub, gs)
            tx1 = _cheb_step4_tc(n_pad, spa[0], spa[1], spb[0], spb[1], dis)
            txs = [h, tx1]

        if li == n_layers - 1:
            wl, bl = params["linear"]
            out = _final_tc(n_pad, cout, txs[0], txs[1], ws[0], ws[1], bias,
                            wl, bl.reshape(1, -1))
            return out[:n]

        cpad_out = _pad16(cout)
        split_u = cpad_out > 32
        res = _layer_end_tc(n_pad, cpad_in, cout, cpad_out, split_u,
                            txs, ws, bias, dis)
        if split_u:
            h, ua, ub = res
            u = (ua, ub)
        else:
            h, u = res
    return None


# speedup vs baseline: 1.0678x; 1.0678x over previous
import functools

import jax
import jax.numpy as jnp
from jax import lax
from jax.experimental import pallas as pl
from jax.experimental.pallas import tpu as pltpu
from jax.experimental.pallas import tpu_sc as plsc

CHUNK = 128
NTILES = 32
IBLK = 28
ZROWS = 112


def _npad(n):
    rpt = -(-(-(-(n + 1) // 16)) // ZROWS) * ZROWS
    return 16 * rpt


def _sc_mesh():
    return plsc.VectorSubcoreMesh(core_axis_name="c", subcore_axis_name="s")


_SC_PARAMS = pltpu.CompilerParams(use_tc_tiling_on_sc=False)


def _matvec_sc(n_pad, cpad, cpt):
    rpt = n_pad // 16
    nblk = cpt // IBLK
    S = 7 if cpad <= 16 else 4
    ngrp = IBLK // S
    assert IBLK % S == 0

    @functools.partial(
        pl.kernel,
        out_type=jax.ShapeDtypeStruct((2, n_pad, cpad), jnp.float32),
        mesh=_sc_mesh(),
        compiler_params=_SC_PARAMS,
        scratch_types=[
            pltpu.VMEM((2, IBLK, CHUNK), jnp.int32),
            pltpu.VMEM((S, CHUNK, cpad), jnp.float32),
            pltpu.VMEM_SHARED((n_pad, cpad), jnp.float32),
            [pltpu.SemaphoreType.DMA] * S,
        ],
    )
    def mv(u_hbm, gs_hbm, out_hbm, gs_v, rows_v, acc, sems):
        cid = lax.axis_index("c")
        sid = lax.axis_index("s")
        wid = cid * 16 + sid

        def _zfill(r, c):
            for j in range(cpad // 16):
                rows_v[0, r, pl.ds(16 * j, 16)] = jnp.zeros((16,), jnp.float32)
            return c

        lax.fori_loop(0, ZROWS, _zfill, 0)
        base = sid * rpt
        for j in range(rpt // ZROWS):
            pltpu.sync_copy(rows_v.at[0, pl.ds(0, ZROWS)],
                            acc.at[pl.ds(base + j * ZROWS, ZROWS)])
        plsc.subcore_barrier()

        def _block(ob, c):
            pltpu.sync_copy(gs_hbm.at[wid, ob], gs_v)
            for b in range(S):
                pltpu.async_copy(u_hbm.at[gs_v.at[0, b]], rows_v.at[b], sems[b])

            def _grp(p, c2):
                for b in range(S):
                    j = p * S + b
                    pltpu.make_async_copy(u_hbm.at[gs_v.at[0, j]],
                                          rows_v.at[b], sems[b]).wait()
                    pltpu.sync_copy(rows_v.at[b], acc.at[gs_v.at[1, j]], add=True)

                    @pl.when(p < ngrp - 1)
                    def _():
                        pltpu.async_copy(u_hbm.at[gs_v.at[0, j + S]],
                                         rows_v.at[b], sems[b])
                return c2

            lax.fori_loop(0, ngrp, _grp, 0)
            return c

        lax.fori_loop(0, nblk, _block, 0)
        plsc.subcore_barrier()
        pltpu.sync_copy(acc.at[pl.ds(base, rpt)],
                        out_hbm.at[cid, pl.ds(base, rpt)])

    return mv


def _degree_sc(n_pad, cpt):
    rpt = n_pad // 16

    @functools.partial(
        pl.kernel,
        out_type=jax.ShapeDtypeStruct((2, n_pad, 16), jnp.float32),
        mesh=_sc_mesh(),
        compiler_params=_SC_PARAMS,
        scratch_types=[
            pltpu.VMEM((IBLK, CHUNK), jnp.int32),
            pltpu.VMEM((CHUNK, 16), jnp.float32),
            pltpu.VMEM((ZROWS, 16), jnp.float32),
            pltpu.VMEM_SHARED((n_pad, 16), jnp.float32),
            pltpu.SemaphoreType.DMA,
        ],
    )
    def dg(s_hbm, out_hbm, s_v, ones_v, zb_v, acc, sem):
        cid = lax.axis_index("c")
        sid = lax.axis_index("s")
        wid = cid * 16 + sid

        def _ofill(r, c):
            ones_v[r, pl.ds(0, 16)] = jnp.ones((16,), jnp.float32)
            return c

        lax.fori_loop(0, CHUNK, _ofill, 0)

        def _zfill(r, c):
            zb_v[r, pl.ds(0, 16)] = jnp.zeros((16,), jnp.float32)
            return c

        lax.fori_loop(0, ZROWS, _zfill, 0)
        base = sid * rpt
        for j in range(rpt // ZROWS):
            pltpu.sync_copy(zb_v, acc.at[pl.ds(base + j * ZROWS, ZROWS)])
        plsc.subcore_barrier()

        def _outer(ob, c):
            pltpu.sync_copy(s_hbm.at[wid, pl.ds(ob * IBLK, IBLK)], s_v)
            for half in range(2):
                descs = [
                    pltpu.async_copy(ones_v, acc.at[s_v.at[half * (IBLK // 2) + j]],
                                     sem, add=True)
                    for j in range(IBLK // 2)
                ]
                for d in descs:
                    d.wait()
            return c

        lax.fori_loop(0, cpt // IBLK, _outer, 0)
        plsc.subcore_barrier()
        pltpu.sync_copy(acc.at[pl.ds(base, rpt)],
                        out_hbm.at[cid, pl.ds(base, rpt)])

    return dg


def _row_spec(rb, w):
    return pl.BlockSpec((rb, w), lambda i: (i, 0))


def _full_spec(shape):
    return pl.BlockSpec(shape, lambda i: tuple(0 for _ in shape))


def _prep_tc(n_pad, xp, da, db):
    rb = n_pad // 16

    def body(xp_ref, da_ref, db_ref, dis_ref, u0_ref):
        deg = da_ref[:, 0:1] + db_ref[:, 0:1]
        dis = jnp.where(deg > 0.0, lax.rsqrt(jnp.maximum(deg, 1e-30)), 0.0)
        dis_ref[...] = dis
        u0_ref[...] = dis * xp_ref[...]

    return pl.pallas_call(
        body,
        grid=(16,),
        in_specs=[_row_spec(rb, 16)] * 3,
        out_specs=[_row_spec(rb, 1), _row_spec(rb, 16)],
        out_shape=[jax.ShapeDtypeStruct((n_pad, 1), jnp.float32),
                   jax.ShapeDtypeStruct((n_pad, 16), jnp.float32)],
    )(xp, da, db)


def _cheb_step_tc(n_pad, cpad, first, sa, sb, dis, txm2):
    rb = n_pad // 16

    if first:
        def body(sa_ref, sb_ref, d_ref, tx_ref, u_ref):
            m = -(d_ref[...] * (sa_ref[...] + sb_ref[...]))
            tx_ref[...] = m
            u_ref[...] = d_ref[...] * m
        ins = [sa, sb, dis]
        in_specs = [_row_spec(rb, cpad)] * 2 + [_row_spec(rb, 1)]
    else:
        def body(sa_ref, sb_ref, d_ref, t2_ref, tx_ref, u_ref):
            m = -(d_ref[...] * (sa_ref[...] + sb_ref[...]))
            tx = 2.0 * m - t2_ref[...]
            tx_ref[...] = tx
            u_ref[...] = d_ref[...] * tx
        ins = [sa, sb, dis, txm2]
        in_specs = [_row_spec(rb, cpad)] * 2 + [_row_spec(rb, 1), _row_spec(rb, cpad)]

    return pl.pallas_call(
        body,
        grid=(16,),
        in_specs=in_specs,
        out_specs=[_row_spec(rb, cpad)] * 2,
        out_shape=[jax.ShapeDtypeStruct((n_pad, cpad), jnp.float32)] * 2,
    )(*ins)


def _cheb_step4_tc(n_pad, sa0, sb0, sa1, sb1, dis):
    rb = n_pad // 16

    def body(sa0_ref, sb0_ref, sa1_ref, sb1_ref, d_ref, tx_ref):
        s = jnp.concatenate([sa0_ref[...] + sb0_ref[...],
                             sa1_ref[...] + sb1_ref[...]], axis=1)
        tx_ref[...] = -(d_ref[...] * s)

    return pl.pallas_call(
        body,
        grid=(16,),
        in_specs=[_row_spec(rb, 32)] * 4 + [_row_spec(rb, 1)],
        out_specs=_row_spec(rb, 64),
        out_shape=jax.ShapeDtypeStruct((n_pad, 64), jnp.float32),
    )(sa0, sb0, sa1, sb1, dis)


def _layer_end_tc(n_pad, cpad_in, cout, cpad_out, split_u, txs, ws, bias, dis):
    rb = n_pad // 16
    K = len(txs)

    def body(*refs):
        tx_refs = refs[:K]
        w_refs = refs[K:2 * K]
        b_ref = refs[2 * K]
        d_ref = refs[2 * K + 1]
        outs = refs[2 * K + 2:]
        acc = b_ref[...]
        for k in range(K):
            acc = acc + jnp.dot(tx_refs[k][...], w_refs[k][...],
                                preferred_element_type=jnp.float32)
        z = 1.0 / (1.0 + jnp.exp(-acc[:, :cout]))
        ht = jnp.tanh(acc[:, cout:])
        h = jnp.maximum((1.0 - z) * ht, 0.0)
        if cpad_out > cout:
            h = jnp.concatenate(
                [h, jnp.zeros((rb, cpad_out - cout), jnp.float32)], axis=1)
        u = d_ref[...] * h
        outs[0][...] = h
        if split_u:
            outs[1][...] = u[:, :32]
            outs[2][...] = u[:, 32:]
        else:
            outs[1][...] = u

    in_specs = ([_row_spec(rb, cpad_in)] * K
                + [_full_spec((cpad_in, 2 * cout))] * K
                + [_full_spec((1, 2 * cout)), _row_spec(rb, 1)])
    if split_u:
        out_specs = [_row_spec(rb, cpad_out), _row_spec(rb, 32), _row_spec(rb, 32)]
        out_shape = [jax.ShapeDtypeStruct((n_pad, cpad_out), jnp.float32),
                     jax.ShapeDtypeStruct((n_pad, 32), jnp.float32),
                     jax.ShapeDtypeStruct((n_pad, 32), jnp.float32)]
    else:
        out_specs = [_row_spec(rb, cpad_out)] * 2
        out_shape = [jax.ShapeDtypeStruct((n_pad, cpad_out), jnp.float32)] * 2

    return pl.pallas_call(
        body,
        grid=(16,),
        in_specs=in_specs,
        out_specs=out_specs,
        out_shape=out_shape,
    )(*txs, *ws, bias, dis)


def _final_tc(n_pad, cout, tx0, tx1, w0, w1, bias, wl, bl):
    rb = n_pad // 16

    def body(t0, t1, w0_ref, w1_ref, b_ref, wl_ref, bl_ref, out_ref):
        acc = (b_ref[...]
               + jnp.dot(t0[...], w0_ref[...], preferred_element_type=jnp.float32)
               + jnp.dot(t1[...], w1_ref[...], preferred_element_type=jnp.float32))
        z = 1.0 / (1.0 + jnp.exp(-acc[:, :cout]))
        ht = jnp.tanh(acc[:, cout:])
        h = jnp.maximum((1.0 - z) * ht, 0.0)
        logits = jnp.dot(h, wl_ref[...], preferred_element_type=jnp.float32) + bl_ref[...]
        m = jnp.max(logits, axis=1, keepdims=True)
        e = jnp.exp(logits - m)
        out_ref[...] = e / jnp.sum(e, axis=1, keepdims=True)

    nc = wl.shape[1]
    return pl.pallas_call(
        body,
        grid=(16,),
        in_specs=[_row_spec(rb, 64)] * 2
        + [_full_spec((64, 2 * cout))] * 2
        + [_full_spec((1, 2 * cout)), _full_spec((cout, nc)), _full_spec((1, nc))],
        out_specs=_row_spec(rb, nc),
        out_shape=jax.ShapeDtypeStruct((n_pad, nc), jnp.float32),
    )(tx0, tx1, w0, w1, bias, wl, bl)


def _pad16(c):
    return -(-c // 16) * 16


def kernel(x, edge_index, params):
    n = x.shape[0]
    e = edge_index.shape[1]
    n_pad = _npad(n)
    cpt = IBLK * -(-e // (NTILES * CHUNK * IBLK))
    ep = NTILES * cpt * CHUNK

    nblk = cpt // IBLK
    row = edge_index[0]
    col = edge_index[1]
    pad_e = ep - e
    gather_pad = jnp.zeros((pad_e,), jnp.int32)
    scatter_pad = jnp.full((pad_e,), n, jnp.int32)
    g_row = jnp.concatenate([row, gather_pad]).reshape(NTILES, nblk, IBLK, CHUNK)
    s_col = jnp.concatenate([col, scatter_pad]).reshape(NTILES, nblk, IBLK, CHUNK)
    s_row = jnp.concatenate([row, scatter_pad]).reshape(NTILES, cpt, CHUNK)
    gs = jnp.stack([g_row, s_col], axis=2)

    degp = _degree_sc(n_pad, cpt)(s_row)
    xp = jnp.pad(x, ((0, n_pad - n), (0, _pad16(x.shape[1]) - x.shape[1])))
    dis, u = _prep_tc(n_pad, xp, degp[0], degp[1])

    h = xp
    n_layers = len(params["layers"])
    for li, lp in enumerate(params["layers"]):
        K, cin, cout = lp["xz"][0].shape
        cpad_in = _pad16(cin)
        ws = []
        for k in range(K):
            wk = jnp.concatenate([lp["xz"][0][k], lp["xh"][0][k]], axis=1)
            ws.append(jnp.pad(wk, ((0, cpad_in - cin), (0, 0))))
        bias = jnp.concatenate([lp["xz"][1] + lp["hz"][1],
                                lp["xh"][1] + lp["hh"][1]]).reshape(1, 2 * cout)

        if cpad_in <= 32:
            txs = [h]
            for k in range(1, K):
                sp = _matvec_sc(n_pad, cpad_in, cpt)(u, gs)
                tx, u = _cheb_step_tc(n_pad, cpad_in, k == 1, sp[0], sp[1],
                                      dis, None if k == 1 else txs[k - 2])
                txs.append(tx)
        else:
            ua, ub = u
            spa = _matvec_sc(n_pad, 32, cpt)(ua, gs)
            spb = _matvec_sc(n_pad, 32, cpt)(ub, gs)
            tx1 = _cheb_step4_tc(n_pad, spa[0], spa[1], spb[0], spb[1], dis)
            txs = [h, tx1]

        if li == n_layers - 1:
            wl, bl = params["linear"]
            out = _final_tc(n_pad, cout, txs[0], txs[1], ws[0], ws[1], bias,
                            wl, bl.reshape(1, -1))
            return out[:n]

        cpad_out = _pad16(cout)
        split_u = cpad_out > 32
        res = _layer_end_tc(n_pad, cpad_in, cout, cpad_out, split_u,
                            txs, ws, bias, dis)
        if split_u:
            h, ua, ub = res
            u = (ua, ub)
        else:
            h, u = res
    return None


# speedup vs baseline: 1.2210x; 1.1435x over previous
import functools

import jax
import jax.numpy as jnp
from jax import lax
from jax.experimental import pallas as pl
from jax.experimental.pallas import tpu as pltpu
from jax.experimental.pallas import tpu_sc as plsc

CHUNK = 128
NTILES = 32
IBLK = 28
ZROWS = 112


def _npad(n):
    rpt = -(-(-(-(n + 1) // 16)) // ZROWS) * ZROWS
    return 16 * rpt


def _sc_mesh():
    return plsc.VectorSubcoreMesh(core_axis_name="c", subcore_axis_name="s")


_SC_PARAMS = pltpu.CompilerParams(use_tc_tiling_on_sc=False)


def _matvec_sc(n_pad, cpad, cpt):
    rpt = n_pad // 16
    nblk = cpt // IBLK
    S = 7 if cpad <= 16 else 4
    ngrp = IBLK // S
    assert IBLK % S == 0

    @functools.partial(
        pl.kernel,
        out_type=jax.ShapeDtypeStruct((2, n_pad, cpad), jnp.float32),
        mesh=_sc_mesh(),
        compiler_params=_SC_PARAMS,
        scratch_types=[
            pltpu.VMEM((2, IBLK, CHUNK), jnp.int32),
            pltpu.VMEM((S, CHUNK, cpad), jnp.float32),
            pltpu.VMEM_SHARED((n_pad, cpad), jnp.float32),
            [pltpu.SemaphoreType.DMA] * S,
        ],
    )
    def mv(u_hbm, gs_hbm, out_hbm, gs_v, rows_v, acc, sems):
        cid = lax.axis_index("c")
        sid = lax.axis_index("s")
        wid = cid * 16 + sid

        def _zfill(r, c):
            for j in range(cpad // 16):
                rows_v[0, r, pl.ds(16 * j, 16)] = jnp.zeros((16,), jnp.float32)
            return c

        lax.fori_loop(0, ZROWS, _zfill, 0)
        base = sid * rpt
        for j in range(rpt // ZROWS):
            pltpu.sync_copy(rows_v.at[0, pl.ds(0, ZROWS)],
                            acc.at[pl.ds(base + j * ZROWS, ZROWS)])
        plsc.subcore_barrier()

        def _block(ob, c):
            pltpu.sync_copy(gs_hbm.at[wid, ob], gs_v)
            for b in range(S):
                pltpu.async_copy(u_hbm.at[gs_v.at[0, b]], rows_v.at[b], sems[b])

            def _grp(p, c2):
                for b in range(S):
                    j = p * S + b
                    pltpu.make_async_copy(u_hbm.at[gs_v.at[0, j]],
                                          rows_v.at[b], sems[b]).wait()
                    pltpu.sync_copy(rows_v.at[b], acc.at[gs_v.at[1, j]], add=True)

                    @pl.when(p < ngrp - 1)
                    def _():
                        pltpu.async_copy(u_hbm.at[gs_v.at[0, j + S]],
                                         rows_v.at[b], sems[b])
                return c2

            lax.fori_loop(0, ngrp, _grp, 0)
            return c

        lax.fori_loop(0, nblk, _block, 0)
        plsc.subcore_barrier()
        pltpu.sync_copy(acc.at[pl.ds(base, rpt)],
                        out_hbm.at[cid, pl.ds(base, rpt)])

    return mv


def _degree_sc(n_pad, cpt):
    rpt = n_pad // 16

    @functools.partial(
        pl.kernel,
        out_type=jax.ShapeDtypeStruct((2, n_pad, 16), jnp.float32),
        mesh=_sc_mesh(),
        compiler_params=_SC_PARAMS,
        scratch_types=[
            pltpu.VMEM((IBLK, CHUNK), jnp.int32),
            pltpu.VMEM((CHUNK, 16), jnp.float32),
            pltpu.VMEM((ZROWS, 16), jnp.float32),
            pltpu.VMEM_SHARED((n_pad, 16), jnp.float32),
            pltpu.SemaphoreType.DMA,
        ],
    )
    def dg(s_hbm, out_hbm, s_v, ones_v, zb_v, acc, sem):
        cid = lax.axis_index("c")
        sid = lax.axis_index("s")
        wid = cid * 16 + sid

        def _ofill(r, c):
            ones_v[r, pl.ds(0, 16)] = jnp.ones((16,), jnp.float32)
            return c

        lax.fori_loop(0, CHUNK, _ofill, 0)

        def _zfill(r, c):
            zb_v[r, pl.ds(0, 16)] = jnp.zeros((16,), jnp.float32)
            return c

        lax.fori_loop(0, ZROWS, _zfill, 0)
        base = sid * rpt
        for j in range(rpt // ZROWS):
            pltpu.sync_copy(zb_v, acc.at[pl.ds(base + j * ZROWS, ZROWS)])
        plsc.subcore_barrier()

        def _outer(ob, c):
            pltpu.sync_copy(s_hbm.at[wid, pl.ds(ob * IBLK, IBLK)], s_v)
            for half in range(2):
                descs = [
                    pltpu.async_copy(ones_v, acc.at[s_v.at[half * (IBLK // 2) + j]],
                                     sem, add=True)
                    for j in range(IBLK // 2)
                ]
                for d in descs:
                    d.wait()
            return c

        lax.fori_loop(0, cpt // IBLK, _outer, 0)
        plsc.subcore_barrier()
        pltpu.sync_copy(acc.at[pl.ds(base, rpt)],
                        out_hbm.at[cid, pl.ds(base, rpt)])

    return dg


def _row_spec(rb, w):
    return pl.BlockSpec((rb, w), lambda i: (i, 0))


def _part_spec(rb, w, c):
    return pl.BlockSpec((1, rb, w), lambda i, _c=c: (_c, i, 0))


def _full_spec(shape):
    return pl.BlockSpec(shape, lambda i: tuple(0 for _ in shape))


def _prep_tc(n_pad, xp, degp):
    rb = n_pad // 16

    def body(xp_ref, da_ref, db_ref, dis_ref, u0_ref):
        deg = da_ref[0, :, 0:1] + db_ref[0, :, 0:1]
        dis = jnp.where(deg > 0.0, lax.rsqrt(jnp.maximum(deg, 1e-30)), 0.0)
        dis_ref[...] = dis
        u0_ref[...] = dis * xp_ref[...]

    return pl.pallas_call(
        body,
        grid=(16,),
        in_specs=[_row_spec(rb, 16), _part_spec(rb, 16, 0), _part_spec(rb, 16, 1)],
        out_specs=[_row_spec(rb, 1), _row_spec(rb, 16)],
        out_shape=[jax.ShapeDtypeStruct((n_pad, 1), jnp.float32),
                   jax.ShapeDtypeStruct((n_pad, 16), jnp.float32)],
    )(xp, degp, degp)


def _cheb_step_tc(n_pad, cpad, first, sp, dis, txm2):
    rb = n_pad // 16

    if first:
        def body(sa_ref, sb_ref, d_ref, tx_ref, u_ref):
            m = -(d_ref[...] * (sa_ref[0] + sb_ref[0]))
            tx_ref[...] = m
            u_ref[...] = d_ref[...] * m
        ins = [sp, sp, dis]
        in_specs = [_part_spec(rb, cpad, 0), _part_spec(rb, cpad, 1),
                    _row_spec(rb, 1)]
    else:
        def body(sa_ref, sb_ref, d_ref, t2_ref, tx_ref, u_ref):
            m = -(d_ref[...] * (sa_ref[0] + sb_ref[0]))
            tx = 2.0 * m - t2_ref[...]
            tx_ref[...] = tx
            u_ref[...] = d_ref[...] * tx
        ins = [sp, sp, dis, txm2]
        in_specs = [_part_spec(rb, cpad, 0), _part_spec(rb, cpad, 1),
                    _row_spec(rb, 1), _row_spec(rb, cpad)]

    return pl.pallas_call(
        body,
        grid=(16,),
        in_specs=in_specs,
        out_specs=[_row_spec(rb, cpad)] * 2,
        out_shape=[jax.ShapeDtypeStruct((n_pad, cpad), jnp.float32)] * 2,
    )(*ins)


def _cheb_step4_tc(n_pad, spa, spb, dis):
    rb = n_pad // 16

    def body(sa0_ref, sb0_ref, sa1_ref, sb1_ref, d_ref, tx_ref):
        s = jnp.concatenate([sa0_ref[0] + sb0_ref[0],
                             sa1_ref[0] + sb1_ref[0]], axis=1)
        tx_ref[...] = -(d_ref[...] * s)

    return pl.pallas_call(
        body,
        grid=(16,),
        in_specs=[_part_spec(rb, 32, 0), _part_spec(rb, 32, 1),
                  _part_spec(rb, 32, 0), _part_spec(rb, 32, 1),
                  _row_spec(rb, 1)],
        out_specs=_row_spec(rb, 64),
        out_shape=jax.ShapeDtypeStruct((n_pad, 64), jnp.float32),
    )(spa, spa, spb, spb, dis)


def _layer_end_tc(n_pad, cpad_in, cout, cpad_out, split_u, txs, ws, bias, dis):
    rb = n_pad // 16
    K = len(txs)

    def body(*refs):
        tx_refs = refs[:K]
        w_refs = refs[K:2 * K]
        b_ref = refs[2 * K]
        d_ref = refs[2 * K + 1]
        outs = refs[2 * K + 2:]
        acc = b_ref[...]
        for k in range(K):
            acc = acc + jnp.dot(tx_refs[k][...], w_refs[k][...],
                                preferred_element_type=jnp.float32)
        z = 1.0 / (1.0 + jnp.exp(-acc[:, :cout]))
        ht = jnp.tanh(acc[:, cout:])
        h = jnp.maximum((1.0 - z) * ht, 0.0)
        if cpad_out > cout:
            h = jnp.concatenate(
                [h, jnp.zeros((rb, cpad_out - cout), jnp.float32)], axis=1)
        u = d_ref[...] * h
        outs[0][...] = h
        if split_u:
            outs[1][...] = u[:, :32]
            outs[2][...] = u[:, 32:]
        else:
            outs[1][...] = u

    in_specs = ([_row_spec(rb, cpad_in)] * K
                + [_full_spec((cpad_in, 2 * cout))] * K
                + [_full_spec((1, 2 * cout)), _row_spec(rb, 1)])
    if split_u:
        out_specs = [_row_spec(rb, cpad_out), _row_spec(rb, 32), _row_spec(rb, 32)]
        out_shape = [jax.ShapeDtypeStruct((n_pad, cpad_out), jnp.float32),
                     jax.ShapeDtypeStruct((n_pad, 32), jnp.float32),
                     jax.ShapeDtypeStruct((n_pad, 32), jnp.float32)]
    else:
        out_specs = [_row_spec(rb, cpad_out)] * 2
        out_shape = [jax.ShapeDtypeStruct((n_pad, cpad_out), jnp.float32)] * 2

    return pl.pallas_call(
        body,
        grid=(16,),
        in_specs=in_specs,
        out_specs=out_specs,
        out_shape=out_shape,
    )(*txs, *ws, bias, dis)


def _final_tc(n_pad, cout, tx0, tx1, w0, w1, bias, wl, bl):
    rb = n_pad // 16

    def body(t0, t1, w0_ref, w1_ref, b_ref, wl_ref, bl_ref, out_ref):
        acc = (b_ref[...]
               + jnp.dot(t0[...], w0_ref[...], preferred_element_type=jnp.float32)
               + jnp.dot(t1[...], w1_ref[...], preferred_element_type=jnp.float32))
        z = 1.0 / (1.0 + jnp.exp(-acc[:, :cout]))
        ht = jnp.tanh(acc[:, cout:])
        h = jnp.maximum((1.0 - z) * ht, 0.0)
        logits = jnp.dot(h, wl_ref[...], preferred_element_type=jnp.float32) + bl_ref[...]
        m = jnp.max(logits, axis=1, keepdims=True)
        e = jnp.exp(logits - m)
        out_ref[...] = e / jnp.sum(e, axis=1, keepdims=True)

    nc = wl.shape[1]
    return pl.pallas_call(
        body,
        grid=(16,),
        in_specs=[_row_spec(rb, 64)] * 2
        + [_full_spec((64, 2 * cout))] * 2
        + [_full_spec((1, 2 * cout)), _full_spec((cout, nc)), _full_spec((1, nc))],
        out_specs=_row_spec(rb, nc),
        out_shape=jax.ShapeDtypeStruct((n_pad, nc), jnp.float32),
    )(tx0, tx1, w0, w1, bias, wl, bl)


def _pad16(c):
    return -(-c // 16) * 16


def kernel(x, edge_index, params):
    n = x.shape[0]
    e = edge_index.shape[1]
    n_pad = _npad(n)
    cpt = IBLK * -(-e // (NTILES * CHUNK * IBLK))
    ep = NTILES * cpt * CHUNK

    nblk = cpt // IBLK
    row = edge_index[0]
    col = edge_index[1]
    pad_e = ep - e
    gather_pad = jnp.zeros((pad_e,), jnp.int32)
    scatter_pad = jnp.full((pad_e,), n, jnp.int32)
    g_row = jnp.concatenate([row, gather_pad]).reshape(NTILES, nblk, IBLK, CHUNK)
    s_col = jnp.concatenate([col, scatter_pad]).reshape(NTILES, nblk, IBLK, CHUNK)
    s_row = jnp.concatenate([row, scatter_pad]).reshape(NTILES, cpt, CHUNK)
    gs = jnp.stack([g_row, s_col], axis=2)

    degp = _degree_sc(n_pad, cpt)(s_row)
    xp = jnp.pad(x, ((0, n_pad - n), (0, _pad16(x.shape[1]) - x.shape[1])))
    dis, u = _prep_tc(n_pad, xp, degp)

    h = xp
    n_layers = len(params["layers"])
    for li, lp in enumerate(params["layers"]):
        K, cin, cout = lp["xz"][0].shape
        cpad_in = _pad16(cin)
        ws = []
        for k in range(K):
            wk = jnp.concatenate([lp["xz"][0][k], lp["xh"][0][k]], axis=1)
            ws.append(jnp.pad(wk, ((0, cpad_in - cin), (0, 0))))
        bias = jnp.concatenate([lp["xz"][1] + lp["hz"][1],
                                lp["xh"][1] + lp["hh"][1]]).reshape(1, 2 * cout)

        if cpad_in <= 32:
            txs = [h]
            for k in range(1, K):
                sp = _matvec_sc(n_pad, cpad_in, cpt)(u, gs)
                tx, u = _cheb_step_tc(n_pad, cpad_in, k == 1, sp,
                                      dis, None if k == 1 else txs[k - 2])
                txs.append(tx)
        else:
            ua, ub = u
            spa = _matvec_sc(n_pad, 32, cpt)(ua, gs)
            spb = _matvec_sc(n_pad, 32, cpt)(ub, gs)
            tx1 = _cheb_step4_tc(n_pad, spa, spb, dis)
            txs = [h, tx1]

        if li == n_layers - 1:
            wl, bl = params["linear"]
            out = _final_tc(n_pad, cout, txs[0], txs[1], ws[0], ws[1], bias,
                            wl, bl.reshape(1, -1))
            return out[:n]

        cpad_out = _pad16(cout)
        split_u = cpad_out > 32
        res = _layer_end_tc(n_pad, cpad_in, cout, cpad_out, split_u,
                            txs, ws, bias, dis)
        if split_u:
            h, ua, ub = res
            u = (ua, ub)
        else:
            h, u = res
    return None
